# Initial kernel scaffold; baseline (speedup 1.0000x reference)
import jax
import jax.numpy as jnp
from jax.experimental import pallas as pl


def kernel(d, x, ss, vb, npatches, patches, npp, sRef, A, Tarr, meanY, ds, lam2, device):
    raise NotImplementedError("write your pallas kernel here")


# speedup vs baseline: 7.2195x; 7.2195x over previous
import functools

import jax
import jax.numpy as jnp
from jax import lax
from jax.experimental import pallas as pl
from jax.experimental.pallas import tpu as pltpu
from jax.experimental.pallas import tpu_sc as plsc

_NC = 2
_NS = 16
_NW = _NC * _NS
_ROW = 128


def _ssb_matmul(d, x, meanY):
    P, K = d.shape
    NP = x.shape[1]
    BN = 2048

    def body(d_ref, x_ref, my_ref, o_ref):
        o_ref[...] = (
            lax.dot_general(
                d_ref[...], x_ref[...],
                (((1,), (0,)), ((), ())),
                preferred_element_type=jnp.float32,
                precision=lax.Precision.HIGHEST,
            )
            + my_ref[...]
        )

    return pl.pallas_call(
        body,
        grid=(NP // BN,),
        in_specs=[
            pl.BlockSpec((P, K), lambda i: (0, 0)),
            pl.BlockSpec((K, BN), lambda i: (0, i)),
            pl.BlockSpec((1, BN), lambda i: (0, i)),
        ],
        out_specs=pl.BlockSpec((P, BN), lambda i: (0, i)),
        out_shape=jax.ShapeDtypeStruct((P, NP), jnp.float32),
    )(d, x, meanY)


def _sc_scatter(idx2, val2, zeros1d):
    ROWS = idx2.shape[0]
    N = zeros1d.shape[0]
    RPW = ROWS // _NW
    SEG = N // _NS

    mesh = plsc.VectorSubcoreMesh(core_axis_name="c", subcore_axis_name="s")

    @functools.partial(
        pl.kernel,
        out_type=jax.ShapeDtypeStruct((_NC * N,), jnp.float32),
        mesh=mesh,
        scratch_types=[
            pltpu.VMEM((RPW, _ROW), jnp.int32),
            pltpu.VMEM((RPW, _ROW), jnp.float32),
            pltpu.VMEM((SEG,), jnp.float32),
            pltpu.VMEM_SHARED((N,), jnp.float32),
        ],
    )
    def sc_body(idx_hbm, val_hbm, z_hbm, out_hbm, idx_v, val_v, stage_v, acc_sh):
        c = lax.axis_index("c")
        s = lax.axis_index("s")
        wid = s * _NC + c

        pltpu.sync_copy(z_hbm.at[pl.ds(s * SEG, SEG)], stage_v)
        pltpu.sync_copy(stage_v, acc_sh.at[pl.ds(s * SEG, SEG)])
        plsc.subcore_barrier()

        pltpu.sync_copy(idx_hbm.at[pl.ds(wid * RPW, RPW)], idx_v)
        pltpu.sync_copy(val_hbm.at[pl.ds(wid * RPW, RPW)], val_v)

        def body(j, carry):
            pltpu.sync_copy(val_v.at[j], acc_sh.at[idx_v.at[j]], add=True)
            return carry

        lax.fori_loop(0, RPW, body, 0)
        plsc.subcore_barrier()

        pltpu.sync_copy(acc_sh.at[pl.ds(s * SEG, SEG)], stage_v)
        pltpu.sync_copy(stage_v, out_hbm.at[pl.ds(c * N + s * SEG, SEG)])

    return sc_body(idx2, val2, zeros1d)


def _matvec_mse(partials, dsr, nppr, vbr, sRefr, A, Tarr, lam2_11):
    M, N = A.shape
    BN = 2048
    steps = N // BN

    def body(p_ref, ds_ref, npp_ref, vb_ref, sr_ref, a_ref, t_ref, l2_ref,
             o_ref, acc_ref):
        i = pl.program_id(0)
        l2 = l2_ref[0, 0]
        psum = p_ref[0:1, :] + p_ref[1:2, :]
        v = (l2 * ds_ref[...] + psum) / (l2 + npp_ref[...]) * vb_ref[...] \
            + sr_ref[...]
        part = lax.dot_general(
            a_ref[...], v,
            (((1,), (1,)), ((), ())),
            preferred_element_type=jnp.float32,
            precision=lax.Precision.HIGHEST,
        )

        @pl.when(i == 0)
        def _():
            acc_ref[...] = part

        @pl.when(i > 0)
        def _():
            acc_ref[...] = acc_ref[...] + part

        @pl.when(i == steps - 1)
        def _():
            r = acc_ref[...] - t_ref[...]
            o_ref[...] = jnp.sum(r * r).reshape(1, 1) / M

    return pl.pallas_call(
        body,
        grid=(steps,),
        in_specs=[
            pl.BlockSpec((2, BN), lambda i: (0, i)),
            pl.BlockSpec((1, BN), lambda i: (0, i)),
            pl.BlockSpec((1, BN), lambda i: (0, i)),
            pl.BlockSpec((1, BN), lambda i: (0, i)),
            pl.BlockSpec((1, BN), lambda i: (0, i)),
            pl.BlockSpec((M, BN), lambda i: (0, i)),
            pl.BlockSpec((M, 1), lambda i: (0, 0)),
            pl.BlockSpec((1, 1), lambda i: (0, 0)),
        ],
        out_specs=pl.BlockSpec((1, 1), lambda i: (0, 0)),
        out_shape=jax.ShapeDtypeStruct((1, 1), jnp.float32),
        scratch_shapes=[pltpu.VMEM((M, 1), jnp.float32)],
    )(partials, dsr, nppr, vbr, sRefr, A, Tarr, lam2_11)


def kernel(d, x, ss, vb, npatches, patches, npp, sRef, A, Tarr, meanY, ds,
           lam2, device):
    P = d.shape[0]
    NP = x.shape[1]
    N = ss.shape[0]
    M = A.shape[0]

    ssb = _ssb_matmul(d, x, meanY)

    idx2 = patches.reshape(P * NP // _ROW, _ROW)
    val2 = ssb.reshape(P * NP // _ROW, _ROW)
    partials = _sc_scatter(idx2, val2, ss.reshape(N))
    partials = partials.reshape(_NC, N)

    loss = _matvec_mse(
        partials,
        ds.reshape(1, N),
        npp.reshape(1, N),
        vb.reshape(1, N),
        sRef.reshape(1, N),
        A,
        Tarr,
        lam2.reshape(1, 1),
    )
    return loss.reshape(())


# speedup vs baseline: 8.2522x; 1.1430x over previous
import functools

import jax
import jax.numpy as jnp
from jax import lax
from jax.experimental import pallas as pl
from jax.experimental.pallas import tpu as pltpu
from jax.experimental.pallas import tpu_sc as plsc

_NC = 2
_NS = 16
_NW = _NC * _NS
_ROW = 128


def _ssb_matmul(d, x, meanY):
    P, K = d.shape
    NP = x.shape[1]
    BN = 2048

    def body(d_ref, x_ref, my_ref, o_ref):
        o_ref[...] = (
            lax.dot_general(
                d_ref[...], x_ref[...],
                (((1,), (0,)), ((), ())),
                preferred_element_type=jnp.float32,
                precision=lax.Precision.HIGHEST,
            )
            + my_ref[...]
        )

    return pl.pallas_call(
        body,
        grid=(NP // BN,),
        in_specs=[
            pl.BlockSpec((P, K), lambda i: (0, 0)),
            pl.BlockSpec((K, BN), lambda i: (0, i)),
            pl.BlockSpec((1, BN), lambda i: (0, i)),
        ],
        out_specs=pl.BlockSpec((P, BN), lambda i: (0, i)),
        out_shape=jax.ShapeDtypeStruct((P, NP), jnp.float32),
    )(d, x, meanY)


def _sc_scatter(idx2, val2, zeros1d):
    ROWS = idx2.shape[0]
    N = zeros1d.shape[0]
    RPW = ROWS // _NW
    SEG = N // _NS

    mesh = plsc.VectorSubcoreMesh(core_axis_name="c", subcore_axis_name="s")

    @functools.partial(
        pl.kernel,
        out_type=jax.ShapeDtypeStruct((_NC * N,), jnp.float32),
        mesh=mesh,
        scratch_types=[
            pltpu.VMEM((RPW, _ROW), jnp.int32),
            pltpu.VMEM((RPW, _ROW), jnp.float32),
            pltpu.VMEM((SEG,), jnp.float32),
            pltpu.VMEM_SHARED((N,), jnp.float32),
            pltpu.SemaphoreType.DMA,
        ],
    )
    def sc_body(idx_hbm, val_hbm, z_hbm, out_hbm, idx_v, val_v, stage_v, acc_sh,
                sem):
        c = lax.axis_index("c")
        s = lax.axis_index("s")
        wid = s * _NC + c

        pltpu.sync_copy(z_hbm.at[pl.ds(s * SEG, SEG)], stage_v)
        pltpu.sync_copy(stage_v, acc_sh.at[pl.ds(s * SEG, SEG)])
        plsc.subcore_barrier()

        pltpu.sync_copy(idx_hbm.at[pl.ds(wid * RPW, RPW)], idx_v)
        pltpu.sync_copy(val_hbm.at[pl.ds(wid * RPW, RPW)], val_v)

        def body(j, carry):
            pltpu.async_copy(val_v.at[j], acc_sh.at[idx_v.at[j]], sem,
                             add=True)
            return carry

        lax.fori_loop(0, RPW, body, 0)
        pltpu.make_async_copy(
            val_hbm.at[pl.ds(wid * RPW, RPW)], val_v, sem
        ).wait()
        plsc.subcore_barrier()

        pltpu.sync_copy(acc_sh.at[pl.ds(s * SEG, SEG)], stage_v)
        pltpu.sync_copy(stage_v, out_hbm.at[pl.ds(c * N + s * SEG, SEG)])

    return sc_body(idx2, val2, zeros1d)


def _matvec_mse(partials, dsr, nppr, vbr, sRefr, A, Tarr, lam2_11):
    M, N = A.shape
    BN = 4096
    steps = N // BN

    def body(p_ref, ds_ref, npp_ref, vb_ref, sr_ref, a_ref, t_ref, l2_ref,
             o_ref, acc_ref):
        i = pl.program_id(0)
        l2 = l2_ref[0, 0]
        psum = p_ref[0:1, :] + p_ref[1:2, :]
        v = (l2 * ds_ref[...] + psum) / (l2 + npp_ref[...]) * vb_ref[...] \
            + sr_ref[...]
        part = lax.dot_general(
            a_ref[...], v,
            (((1,), (1,)), ((), ())),
            preferred_element_type=jnp.float32,
            precision=lax.Precision.HIGHEST,
        )

        @pl.when(i == 0)
        def _():
            acc_ref[...] = part

        @pl.when(i > 0)
        def _():
            acc_ref[...] = acc_ref[...] + part

        @pl.when(i == steps - 1)
        def _():
            r = acc_ref[...] - t_ref[...]
            o_ref[...] = jnp.sum(r * r).reshape(1, 1) / M

    return pl.pallas_call(
        body,
        grid=(steps,),
        in_specs=[
            pl.BlockSpec((2, BN), lambda i: (0, i)),
            pl.BlockSpec((1, BN), lambda i: (0, i)),
            pl.BlockSpec((1, BN), lambda i: (0, i)),
            pl.BlockSpec((1, BN), lambda i: (0, i)),
            pl.BlockSpec((1, BN), lambda i: (0, i)),
            pl.BlockSpec((M, BN), lambda i: (0, i)),
            pl.BlockSpec((M, 1), lambda i: (0, 0)),
            pl.BlockSpec((1, 1), lambda i: (0, 0)),
        ],
        out_specs=pl.BlockSpec((1, 1), lambda i: (0, 0)),
        out_shape=jax.ShapeDtypeStruct((1, 1), jnp.float32),
        scratch_shapes=[pltpu.VMEM((M, 1), jnp.float32)],
    )(partials, dsr, nppr, vbr, sRefr, A, Tarr, lam2_11)


def kernel(d, x, ss, vb, npatches, patches, npp, sRef, A, Tarr, meanY, ds,
           lam2, device):
    P = d.shape[0]
    NP = x.shape[1]
    N = ss.shape[0]
    M = A.shape[0]

    ssb = _ssb_matmul(d, x, meanY)

    idx2 = patches.reshape(P * NP // _ROW, _ROW)
    val2 = ssb.reshape(P * NP // _ROW, _ROW)
    partials = _sc_scatter(idx2, val2, ss.reshape(N))
    partials = partials.reshape(_NC, N)

    loss = _matvec_mse(
        partials,
        ds.reshape(1, N),
        npp.reshape(1, N),
        vb.reshape(1, N),
        sRef.reshape(1, N),
        A,
        Tarr,
        lam2.reshape(1, 1),
    )
    return loss.reshape(())


# speedup vs baseline: 8.6480x; 1.0480x over previous
import functools

import jax
import jax.numpy as jnp
from jax import lax
from jax.experimental import pallas as pl
from jax.experimental.pallas import tpu as pltpu
from jax.experimental.pallas import tpu_sc as plsc

_NC = 2
_NS = 16
_NW = _NC * _NS
_ROW = 128


def _ssb_matmul(d, x, meanY):
    P, K = d.shape
    NP = x.shape[1]
    BN = 2048

    def body(d_ref, x_ref, my_ref, o_ref):
        o_ref[...] = (
            lax.dot_general(
                d_ref[...], x_ref[...],
                (((1,), (0,)), ((), ())),
                preferred_element_type=jnp.float32,
            )
            + my_ref[...]
        )

    return pl.pallas_call(
        body,
        grid=(NP // BN,),
        in_specs=[
            pl.BlockSpec((P, K), lambda i: (0, 0)),
            pl.BlockSpec((K, BN), lambda i: (0, i)),
            pl.BlockSpec((1, BN), lambda i: (0, i)),
        ],
        out_specs=pl.BlockSpec((P, BN), lambda i: (0, i)),
        out_shape=jax.ShapeDtypeStruct((P, NP), jnp.float32),
    )(d, x, meanY)


def _sc_scatter(idx2, val2, zeros1d):
    ROWS = idx2.shape[0]
    N = zeros1d.shape[0]
    RPW = ROWS // _NW
    SEG = N // _NS

    mesh = plsc.VectorSubcoreMesh(core_axis_name="c", subcore_axis_name="s")

    @functools.partial(
        pl.kernel,
        out_type=jax.ShapeDtypeStruct((_NC * N,), jnp.float32),
        mesh=mesh,
        scratch_types=[
            pltpu.VMEM((RPW, _ROW), jnp.int32),
            pltpu.VMEM((RPW, _ROW), jnp.float32),
            pltpu.VMEM((SEG,), jnp.float32),
            pltpu.VMEM_SHARED((N,), jnp.float32),
            pltpu.SemaphoreType.DMA,
            pltpu.SemaphoreType.DMA,
        ],
    )
    def sc_body(idx_hbm, val_hbm, z_hbm, out_hbm, idx_v, val_v, stage_v, acc_sh,
                sem, load_sem):
        c = lax.axis_index("c")
        s = lax.axis_index("s")
        wid = s * _NC + c

        idx_load = pltpu.async_copy(
            idx_hbm.at[pl.ds(wid * RPW, RPW)], idx_v, load_sem)
        val_load = pltpu.async_copy(
            val_hbm.at[pl.ds(wid * RPW, RPW)], val_v, load_sem)

        pltpu.sync_copy(z_hbm.at[pl.ds(s * SEG, SEG)], stage_v)
        pltpu.sync_copy(stage_v, acc_sh.at[pl.ds(s * SEG, SEG)])
        plsc.subcore_barrier()

        idx_load.wait()
        val_load.wait()

        def body(j, carry):
            pltpu.async_copy(val_v.at[j], acc_sh.at[idx_v.at[j]], sem,
                             add=True)
            return carry

        lax.fori_loop(0, RPW, body, 0)
        pltpu.make_async_copy(
            val_hbm.at[pl.ds(wid * RPW, RPW)], val_v, sem
        ).wait()
        plsc.subcore_barrier()

        pltpu.sync_copy(acc_sh.at[pl.ds(s * SEG, SEG)], stage_v)
        pltpu.sync_copy(stage_v, out_hbm.at[pl.ds(c * N + s * SEG, SEG)])

    return sc_body(idx2, val2, zeros1d)


def _matvec_mse(partials, dsr, nppr, vbr, sRefr, A, Tarr, lam2_11):
    M, N = A.shape
    BN = 4096
    steps = N // BN

    def body(p_ref, ds_ref, npp_ref, vb_ref, sr_ref, a_ref, t_ref, l2_ref,
             o_ref, acc_ref):
        i = pl.program_id(0)
        l2 = l2_ref[0, 0]
        psum = p_ref[0:1, :] + p_ref[1:2, :]
        v = (l2 * ds_ref[...] + psum) / (l2 + npp_ref[...]) * vb_ref[...] \
            + sr_ref[...]
        part = lax.dot_general(
            a_ref[...], v,
            (((1,), (1,)), ((), ())),
            preferred_element_type=jnp.float32,
            precision=lax.Precision.HIGHEST,
        )

        @pl.when(i == 0)
        def _():
            acc_ref[...] = part

        @pl.when(i > 0)
        def _():
            acc_ref[...] = acc_ref[...] + part

        @pl.when(i == steps - 1)
        def _():
            r = acc_ref[...] - t_ref[...]
            o_ref[...] = jnp.sum(r * r).reshape(1, 1) / M

    return pl.pallas_call(
        body,
        grid=(steps,),
        in_specs=[
            pl.BlockSpec((2, BN), lambda i: (0, i)),
            pl.BlockSpec((1, BN), lambda i: (0, i)),
            pl.BlockSpec((1, BN), lambda i: (0, i)),
            pl.BlockSpec((1, BN), lambda i: (0, i)),
            pl.BlockSpec((1, BN), lambda i: (0, i)),
            pl.BlockSpec((M, BN), lambda i: (0, i)),
            pl.BlockSpec((M, 1), lambda i: (0, 0)),
            pl.BlockSpec((1, 1), lambda i: (0, 0)),
        ],
        out_specs=pl.BlockSpec((1, 1), lambda i: (0, 0)),
        out_shape=jax.ShapeDtypeStruct((1, 1), jnp.float32),
        scratch_shapes=[pltpu.VMEM((M, 1), jnp.float32)],
    )(partials, dsr, nppr, vbr, sRefr, A, Tarr, lam2_11)


def kernel(d, x, ss, vb, npatches, patches, npp, sRef, A, Tarr, meanY, ds,
           lam2, device):
    P = d.shape[0]
    NP = x.shape[1]
    N = ss.shape[0]
    M = A.shape[0]

    ssb = _ssb_matmul(d, x, meanY)

    idx2 = patches.reshape(P * NP // _ROW, _ROW)
    val2 = ssb.reshape(P * NP // _ROW, _ROW)
    partials = _sc_scatter(idx2, val2, ss.reshape(N))
    partials = partials.reshape(_NC, N)

    loss = _matvec_mse(
        partials,
        ds.reshape(1, N),
        npp.reshape(1, N),
        vb.reshape(1, N),
        sRef.reshape(1, N),
        A,
        Tarr,
        lam2.reshape(1, 1),
    )
    return loss.reshape(())
